# baseline (speedup 1.0000x reference)
import functools

import jax
import jax.numpy as jnp
from jax import lax
from jax.experimental import pallas as pl
from jax.experimental.pallas import tpu as pltpu
from jax.experimental.pallas import tpu_sc as plsc

_N = 10000
_E = 320000
_D = 128

_NC = 2
_NS = 16
_NW = _NC * _NS
_NP = 10240
_NB = 1024
_NBLK = _NP // _NB
_EP = 327680
_ZROWS = _NP // _NS

_TILE_E = _EP // _NW
_CB = 1024
_CHUNKS = _TILE_E // _CB
_SUB = 128
_STAGE = 2
_SROWS = _TILE_E // _STAGE // 128
_NPAIR = _SROWS // 2

_DEG_CB = 2048
_DEG_ROWS = _DEG_CB // 128
_DEG_CHUNKS = _TILE_E // _DEG_CB

_MESH = dict(core_axis_name="c", subcore_axis_name="s",
             num_cores=_NC, num_subcores=_NS)


def _splat(vec16, lane):
    idx = jnp.full((16, 1), lane, dtype=jnp.int32)
    return lax.gather(
        vec16, idx,
        dimension_numbers=lax.GatherDimensionNumbers(
            offset_dims=(), collapsed_slice_dims=(0,), start_index_map=(0,)),
        slice_sizes=(1,),
        mode=lax.GatherScatterMode.PROMISE_IN_BOUNDS)


@functools.partial(
    pl.kernel,
    out_type=jax.ShapeDtypeStruct((_NC * _NP,), jnp.float32),
    mesh=plsc.VectorSubcoreMesh(**_MESH),
    scratch_types=[
        pltpu.VMEM((_DEG_ROWS, 128), jnp.int32),
        pltpu.VMEM((_DEG_ROWS, 128), jnp.float32),
        pltpu.VMEM((_ZROWS,), jnp.float32),
        pltpu.VMEM_SHARED((_NP,), jnp.float32),
    ],
)
def _deg(col_hbm, ew_hbm, out_hbm, col_v, ew_v, zb, dacc):
    c = lax.axis_index("c")
    s = lax.axis_index("s")
    w = c * _NS + s

    def _z(i, carry):
        zb[pl.ds(i * 16, 16)] = jnp.zeros((16,), jnp.float32)
        return carry
    lax.fori_loop(0, _ZROWS // 16, _z, 0)
    pltpu.sync_copy(zb, dacc.at[pl.ds(pl.multiple_of(s * _ZROWS, 8), _ZROWS)])
    plsc.subcore_barrier()

    def _chunk(k, carry):
        b128 = pl.multiple_of((w * _TILE_E + k * _DEG_CB) // 128, 8)
        pltpu.sync_copy(col_hbm.at[pl.ds(b128, _DEG_ROWS)], col_v)
        pltpu.sync_copy(ew_hbm.at[pl.ds(b128, _DEG_ROWS)], ew_v)

        def _sub(j, carry2):
            pltpu.sync_copy(ew_v.at[j], dacc.at[col_v.at[j]], add=True)
            return carry2
        return lax.fori_loop(0, _DEG_ROWS, _sub, carry)
    lax.fori_loop(0, _DEG_CHUNKS, _chunk, 0)

    plsc.subcore_barrier()

    @pl.when(s == 0)
    def _():
        pltpu.sync_copy(dacc,
                        out_hbm.at[pl.ds(pl.multiple_of(c * _NP, 8), _NP)])


@functools.partial(
    pl.kernel,
    out_type=jax.ShapeDtypeStruct((_NC * _NP, _D), jnp.float32),
    mesh=plsc.VectorSubcoreMesh(**_MESH),
    scratch_types=[
        pltpu.VMEM((2 * _SUB, _D), jnp.float32),
        pltpu.VMEM((_SROWS, 128), jnp.int32),
        pltpu.VMEM((_SROWS, 128), jnp.int32),
        pltpu.VMEM((_SROWS, 128), jnp.float32),
        pltpu.VMEM_SHARED((_NP, _D), jnp.float32),
        pltpu.SemaphoreType.DMA,
        pltpu.SemaphoreType.DMA,
        pltpu.SemaphoreType.DMA,
        pltpu.SemaphoreType.DMA,
    ],
)
def _agg(g_hbm, row_hbm, col_hbm, ew_hbm, out_hbm,
         rows_v, idx_v, col_v, ew_v, acc, gsA, gsB, ssA, ssB):
    c = lax.axis_index("c")
    s = lax.axis_index("s")
    w = c * _NS + s
    nk = _D // 16

    def _z(i, carry):
        for kk in range(nk):
            rows_v[i, pl.ds(kk * 16, 16)] = jnp.zeros((16,), jnp.float32)
        return carry
    lax.fori_loop(0, 2 * _SUB, _z, 0)
    off = pl.multiple_of(s * _ZROWS, 8)
    zoff = 0
    while zoff < _ZROWS:
        zn = min(2 * _SUB, _ZROWS - zoff)
        pltpu.sync_copy(rows_v.at[pl.ds(0, zn)],
                        acc.at[pl.ds(pl.multiple_of(off + zoff, 8), zn)])
        zoff += zn
    plsc.subcore_barrier()

    def _gath(sub, buf, sem):
        pltpu.async_copy(g_hbm.at[idx_v.at[sub]],
                         rows_v.at[pl.ds(buf * _SUB, _SUB)], sem)

    def _wait_gath(buf, sem):
        pltpu.make_async_copy(g_hbm.at[pl.ds(0, _SUB)],
                              rows_v.at[pl.ds(buf * _SUB, _SUB)], sem).wait()

    def _scale(sub, buf):
        def _grp(g, carry):
            ew16 = ew_v[sub, pl.ds(g * 16, 16)]
            eb = buf * _SUB + g * 16
            for jj in range(16):
                sc = _splat(ew16, jj)
                for kk in range(nk):
                    v = rows_v[eb + jj, pl.ds(kk * 16, 16)]
                    rows_v[eb + jj, pl.ds(kk * 16, 16)] = v * sc
            return carry
        lax.fori_loop(0, _SUB // 16, _grp, 0)

    def _scat(sub, buf, sem):
        pltpu.async_copy(rows_v.at[pl.ds(buf * _SUB, _SUB)],
                         acc.at[col_v.at[sub]], sem, add=True)

    def _wait_scat(buf, sem):
        pltpu.make_async_copy(rows_v.at[pl.ds(buf * _SUB, _SUB)],
                              acc.at[pl.ds(0, _SUB)], sem).wait()

    for stage in range(_STAGE):
        b128 = pl.multiple_of((w * _TILE_E) // 128 + stage * _SROWS, 8)
        pltpu.sync_copy(row_hbm.at[pl.ds(b128, _SROWS)], idx_v)
        pltpu.sync_copy(col_hbm.at[pl.ds(b128, _SROWS)], col_v)
        pltpu.sync_copy(ew_hbm.at[pl.ds(b128, _SROWS)], ew_v)

        _gath(0, 0, gsA)
        _gath(1, 1, gsB)

        def _pair(j, carry):
            _wait_gath(0, gsA)
            _scale(2 * j, 0)
            _scat(2 * j, 0, ssA)

            _wait_gath(1, gsB)
            _scale(2 * j + 1, 1)

            @pl.when(j < _NPAIR - 1)
            def _():
                _wait_scat(0, ssA)
                _gath(2 * j + 2, 0, gsA)

            _scat(2 * j + 1, 1, ssB)

            @pl.when(j < _NPAIR - 1)
            def _():
                _wait_scat(1, ssB)
                _gath(2 * j + 3, 1, gsB)
            return carry
        lax.fori_loop(0, _NPAIR, _pair, 0)
        _wait_scat(0, ssA)
        _wait_scat(1, ssB)

    plsc.subcore_barrier()
    pltpu.sync_copy(
        acc.at[pl.ds(off, _ZROWS)],
        out_hbm.at[pl.ds(pl.multiple_of(c * _NP + s * _ZROWS, 8), _ZROWS)])


def _tc1(x_pad, degp):
    def body(x_ref, d_ref, xs_ref, dis_ref):
        deg = d_ref[0, :] + d_ref[1, :] + 1.0
        dis = lax.rsqrt(deg)
        xs_ref[...] = x_ref[...] * dis[:, None]
        dis_ref[...] = dis
    return pl.pallas_call(
        body,
        grid=(_NBLK,),
        in_specs=[
            pl.BlockSpec((_NB, _D), lambda i: (i, 0)),
            pl.BlockSpec((2, _NB), lambda i: (0, i)),
        ],
        out_specs=[
            pl.BlockSpec((_NB, _D), lambda i: (i, 0)),
            pl.BlockSpec((_NB,), lambda i: (i,)),
        ],
        out_shape=[
            jax.ShapeDtypeStruct((_NP, _D), jnp.float32),
            jax.ShapeDtypeStruct((_NP,), jnp.float32),
        ],
    )(x_pad, degp)


def _tc2(sx, xs, dis, b1, W1):
    def body(sx_ref, xs_ref, dis_ref, b1_ref, w_ref, t_ref):
        dis = dis_ref[...]
        u = sx_ref[0] + sx_ref[1] + xs_ref[...]
        a = dis[:, None] * jnp.dot(u, w_ref[...],
                                   preferred_element_type=jnp.float32)
        a = a + b1_ref[...][None, :]
        t_ref[...] = jnp.maximum(a, 0.0) * dis[:, None]
    return pl.pallas_call(
        body,
        grid=(_NBLK,),
        in_specs=[
            pl.BlockSpec((2, _NB, _D), lambda i: (0, i, 0)),
            pl.BlockSpec((_NB, _D), lambda i: (i, 0)),
            pl.BlockSpec((_NB,), lambda i: (i,)),
            pl.BlockSpec((_D,), lambda i: (0,)),
            pl.BlockSpec((_D, _D), lambda i: (0, 0)),
        ],
        out_specs=pl.BlockSpec((_NB, _D), lambda i: (i, 0)),
        out_shape=jax.ShapeDtypeStruct((_NP, _D), jnp.float32),
    )(sx, xs, dis, b1, W1)


def _tc3(st, t, dis, b2, W2):
    def body(st_ref, t_ref, dis_ref, b2_ref, w_ref, o_ref):
        dis = dis_ref[...]
        v = st_ref[0] + st_ref[1] + t_ref[...]
        a = dis[:, None] * jnp.dot(v, w_ref[...],
                                   preferred_element_type=jnp.float32)
        a = a + b2_ref[...][None, :]
        m = jnp.max(a, axis=1, keepdims=True)
        lse = jnp.log(jnp.sum(jnp.exp(a - m), axis=1, keepdims=True)) + m
        o_ref[...] = a - lse
    return pl.pallas_call(
        body,
        grid=(_NBLK,),
        in_specs=[
            pl.BlockSpec((2, _NB, _D), lambda i: (0, i, 0)),
            pl.BlockSpec((_NB, _D), lambda i: (i, 0)),
            pl.BlockSpec((_NB,), lambda i: (i,)),
            pl.BlockSpec((64,), lambda i: (0,)),
            pl.BlockSpec((_D, 64), lambda i: (0, 0)),
        ],
        out_specs=pl.BlockSpec((_NB, 64), lambda i: (i, 0)),
        out_shape=jax.ShapeDtypeStruct((_NP, 64), jnp.float32),
    )(st, t, dis, b2, W2)


def kernel(x, edge_index, edge_weight, W1, b1, W2, b2):
    row = edge_index[0]
    col = edge_index[1]
    pad = _EP - _E
    spread = (jnp.arange(pad, dtype=jnp.int32) * 13) % _N
    rowp = jnp.concatenate([row, spread]).reshape(_EP // 128, 128)
    colp = jnp.concatenate([col, spread]).reshape(_EP // 128, 128)
    ewp = jnp.concatenate(
        [edge_weight, jnp.zeros((pad,), jnp.float32)]).reshape(_EP // 128, 128)
    x_pad = jnp.pad(x, ((0, _NP - _N), (0, 0)))

    degp = _deg(colp, ewp)
    xs, dis = _tc1(x_pad, degp.reshape(2, _NP))
    sx = _agg(xs, rowp, colp, ewp)
    t = _tc2(sx.reshape(2, _NP, _D), xs, dis, b1, W1)
    st = _agg(t, rowp, colp, ewp)
    out_pad = _tc3(st.reshape(2, _NP, _D), t, dis, b2, W2)
    return out_pad[:_N]


# speedup vs baseline: 1.0513x; 1.0513x over previous
import functools

import jax
import jax.numpy as jnp
from jax import lax
from jax.experimental import pallas as pl
from jax.experimental.pallas import tpu as pltpu
from jax.experimental.pallas import tpu_sc as plsc

_N = 10000
_E = 320000
_D = 128

_NC = 2
_NS = 16
_NW = _NC * _NS
_NP = 10240
_NB = 1024
_NBLK = _NP // _NB
_EP = 327680
_ZROWS = _NP // _NS

_TILE_E = _EP // _NW
_CB = 1024
_CHUNKS = _TILE_E // _CB
_SUB = 128
_STAGE = 2
_SROWS = _TILE_E // _STAGE // 128
_NPAIR = _SROWS // 2

_DEG_CB = 2048
_DEG_ROWS = _DEG_CB // 128
_DEG_CHUNKS = _TILE_E // _DEG_CB

_MESH = dict(core_axis_name="c", subcore_axis_name="s",
             num_cores=_NC, num_subcores=_NS)


def _splat(vec16, lane):
    idx = jnp.full((16, 1), lane, dtype=jnp.int32)
    return lax.gather(
        vec16, idx,
        dimension_numbers=lax.GatherDimensionNumbers(
            offset_dims=(), collapsed_slice_dims=(0,), start_index_map=(0,)),
        slice_sizes=(1,),
        mode=lax.GatherScatterMode.PROMISE_IN_BOUNDS)


@functools.partial(
    pl.kernel,
    out_type=jax.ShapeDtypeStruct((_NC * _NP,), jnp.float32),
    mesh=plsc.VectorSubcoreMesh(**_MESH),
    scratch_types=[
        pltpu.VMEM((_DEG_CB,), jnp.int32),
        pltpu.VMEM((_DEG_CB,), jnp.float32),
        pltpu.VMEM((_ZROWS,), jnp.float32),
        pltpu.VMEM_SHARED((_NP,), jnp.float32),
    ],
)
def _deg(col_hbm, ew_hbm, out_hbm, col_v, ew_v, zb, dacc):
    c = lax.axis_index("c")
    s = lax.axis_index("s")
    w = c * _NS + s

    def _z(i, carry):
        zb[pl.ds(i * 16, 16)] = jnp.zeros((16,), jnp.float32)
        return carry
    lax.fori_loop(0, _ZROWS // 16, _z, 0)
    pltpu.sync_copy(zb, dacc.at[pl.ds(pl.multiple_of(s * _ZROWS, 8), _ZROWS)])
    plsc.subcore_barrier()

    def _chunk(k, carry):
        base = pl.multiple_of(w * _TILE_E + k * _DEG_CB, 8)
        pltpu.sync_copy(col_hbm.at[pl.ds(base, _DEG_CB)], col_v)
        pltpu.sync_copy(ew_hbm.at[pl.ds(base, _DEG_CB)], ew_v)

        pltpu.sync_copy(ew_v, dacc.at[col_v], add=True)
        return carry
    lax.fori_loop(0, _DEG_CHUNKS, _chunk, 0)

    plsc.subcore_barrier()

    @pl.when(s == 0)
    def _():
        pltpu.sync_copy(dacc,
                        out_hbm.at[pl.ds(pl.multiple_of(c * _NP, 8), _NP)])


@functools.partial(
    pl.kernel,
    out_type=jax.ShapeDtypeStruct((_NC * _NP, _D), jnp.float32),
    mesh=plsc.VectorSubcoreMesh(**_MESH),
    scratch_types=[
        pltpu.VMEM((2 * _SUB, _D), jnp.float32),
        pltpu.VMEM((_SROWS, 128), jnp.int32),
        pltpu.VMEM((_SROWS, 128), jnp.int32),
        pltpu.VMEM((_SROWS, 128), jnp.float32),
        pltpu.VMEM_SHARED((_NP, _D), jnp.float32),
        pltpu.SemaphoreType.DMA,
        pltpu.SemaphoreType.DMA,
    ],
)
def _agg(g_hbm, row_hbm, col_hbm, ew_hbm, out_hbm,
         rows_v, idx_v, col_v, ew_v, acc, gsA, gsB):
    c = lax.axis_index("c")
    s = lax.axis_index("s")
    w = c * _NS + s
    nk = _D // 16

    def _z(i, carry):
        for kk in range(nk):
            rows_v[i, pl.ds(kk * 16, 16)] = jnp.zeros((16,), jnp.float32)
        return carry
    lax.fori_loop(0, 2 * _SUB, _z, 0)
    off = pl.multiple_of(s * _ZROWS, 8)
    zoff = 0
    while zoff < _ZROWS:
        zn = min(2 * _SUB, _ZROWS - zoff)
        pltpu.sync_copy(rows_v.at[pl.ds(0, zn)],
                        acc.at[pl.ds(pl.multiple_of(off + zoff, 8), zn)])
        zoff += zn
    plsc.subcore_barrier()

    def _gath(sub, buf, sem):
        pltpu.async_copy(g_hbm.at[idx_v.at[sub]],
                         rows_v.at[pl.ds(buf * _SUB, _SUB)], sem)

    def _wait_gath(buf, sem):
        pltpu.make_async_copy(g_hbm.at[pl.ds(0, _SUB)],
                              rows_v.at[pl.ds(buf * _SUB, _SUB)], sem).wait()

    def _scale(sub, buf):
        def _grp(g, carry):
            ew16 = ew_v[sub, pl.ds(g * 16, 16)]
            eb = buf * _SUB + g * 16
            for jj in range(16):
                sc = _splat(ew16, jj)
                for kk in range(nk):
                    v = rows_v[eb + jj, pl.ds(kk * 16, 16)]
                    rows_v[eb + jj, pl.ds(kk * 16, 16)] = v * sc
            return carry
        lax.fori_loop(0, _SUB // 16, _grp, 0)

    def _scat(sub, buf):
        pltpu.sync_copy(rows_v.at[pl.ds(buf * _SUB, _SUB)],
                        acc.at[col_v.at[sub]], add=True)

    for stage in range(_STAGE):
        b128 = pl.multiple_of((w * _TILE_E) // 128 + stage * _SROWS, 8)
        pltpu.sync_copy(row_hbm.at[pl.ds(b128, _SROWS)], idx_v)
        pltpu.sync_copy(col_hbm.at[pl.ds(b128, _SROWS)], col_v)
        pltpu.sync_copy(ew_hbm.at[pl.ds(b128, _SROWS)], ew_v)

        _gath(0, 0, gsA)
        _gath(1, 1, gsB)

        def _pair(j, carry):
            _wait_gath(0, gsA)
            _scale(2 * j, 0)
            _scat(2 * j, 0)

            @pl.when(j < _NPAIR - 1)
            def _():
                _gath(2 * j + 2, 0, gsA)

            _wait_gath(1, gsB)
            _scale(2 * j + 1, 1)
            _scat(2 * j + 1, 1)

            @pl.when(j < _NPAIR - 1)
            def _():
                _gath(2 * j + 3, 1, gsB)
            return carry
        lax.fori_loop(0, _NPAIR, _pair, 0)

    plsc.subcore_barrier()
    pltpu.sync_copy(
        acc.at[pl.ds(off, _ZROWS)],
        out_hbm.at[pl.ds(pl.multiple_of(c * _NP + s * _ZROWS, 8), _ZROWS)])


def _tc1(x_pad, degp):
    def body(x_ref, d_ref, xs_ref, dis_ref):
        deg = d_ref[0, :] + d_ref[1, :] + 1.0
        dis = lax.rsqrt(deg)
        xs_ref[...] = x_ref[...] * dis[:, None]
        dis_ref[...] = dis
    return pl.pallas_call(
        body,
        grid=(_NBLK,),
        in_specs=[
            pl.BlockSpec((_NB, _D), lambda i: (i, 0)),
            pl.BlockSpec((2, _NB), lambda i: (0, i)),
        ],
        out_specs=[
            pl.BlockSpec((_NB, _D), lambda i: (i, 0)),
            pl.BlockSpec((_NB,), lambda i: (i,)),
        ],
        out_shape=[
            jax.ShapeDtypeStruct((_NP, _D), jnp.float32),
            jax.ShapeDtypeStruct((_NP,), jnp.float32),
        ],
    )(x_pad, degp)


def _tc2(sx, xs, dis, b1, W1):
    def body(sx_ref, xs_ref, dis_ref, b1_ref, w_ref, t_ref):
        dis = dis_ref[...]
        u = sx_ref[0] + sx_ref[1] + xs_ref[...]
        a = dis[:, None] * jnp.dot(u, w_ref[...],
                                   preferred_element_type=jnp.float32)
        a = a + b1_ref[...][None, :]
        t_ref[...] = jnp.maximum(a, 0.0) * dis[:, None]
    return pl.pallas_call(
        body,
        grid=(_NBLK,),
        in_specs=[
            pl.BlockSpec((2, _NB, _D), lambda i: (0, i, 0)),
            pl.BlockSpec((_NB, _D), lambda i: (i, 0)),
            pl.BlockSpec((_NB,), lambda i: (i,)),
            pl.BlockSpec((_D,), lambda i: (0,)),
            pl.BlockSpec((_D, _D), lambda i: (0, 0)),
        ],
        out_specs=pl.BlockSpec((_NB, _D), lambda i: (i, 0)),
        out_shape=jax.ShapeDtypeStruct((_NP, _D), jnp.float32),
    )(sx, xs, dis, b1, W1)


def _tc3(st, t, dis, b2, W2):
    def body(st_ref, t_ref, dis_ref, b2_ref, w_ref, o_ref):
        dis = dis_ref[...]
        v = st_ref[0] + st_ref[1] + t_ref[...]
        a = dis[:, None] * jnp.dot(v, w_ref[...],
                                   preferred_element_type=jnp.float32)
        a = a + b2_ref[...][None, :]
        m = jnp.max(a, axis=1, keepdims=True)
        lse = jnp.log(jnp.sum(jnp.exp(a - m), axis=1, keepdims=True)) + m
        o_ref[...] = a - lse
    return pl.pallas_call(
        body,
        grid=(_NBLK,),
        in_specs=[
            pl.BlockSpec((2, _NB, _D), lambda i: (0, i, 0)),
            pl.BlockSpec((_NB, _D), lambda i: (i, 0)),
            pl.BlockSpec((_NB,), lambda i: (i,)),
            pl.BlockSpec((64,), lambda i: (0,)),
            pl.BlockSpec((_D, 64), lambda i: (0, 0)),
        ],
        out_specs=pl.BlockSpec((_NB, 64), lambda i: (i, 0)),
        out_shape=jax.ShapeDtypeStruct((_NP, 64), jnp.float32),
    )(st, t, dis, b2, W2)


def kernel(x, edge_index, edge_weight, W1, b1, W2, b2):
    row = edge_index[0]
    col = edge_index[1]
    pad = _EP - _E
    spread = (jnp.arange(pad, dtype=jnp.int32) * 13) % _N
    rowp = jnp.concatenate([row, spread]).reshape(_EP // 128, 128)
    colp = jnp.concatenate([col, spread]).reshape(_EP // 128, 128)
    ewp = jnp.concatenate(
        [edge_weight, jnp.zeros((pad,), jnp.float32)]).reshape(_EP // 128, 128)
    x_pad = jnp.pad(x, ((0, _NP - _N), (0, 0)))

    degp = _deg(colp.reshape(_EP), ewp.reshape(_EP))
    xs, dis = _tc1(x_pad, degp.reshape(2, _NP))
    sx = _agg(xs, rowp, colp, ewp)
    t = _tc2(sx.reshape(2, _NP, _D), xs, dis, b1, W1)
    st = _agg(t, rowp, colp, ewp)
    out_pad = _tc3(st.reshape(2, _NP, _D), t, dis, b2, W2)
    return out_pad[:_N]
